# Initial kernel scaffold; baseline (speedup 1.0000x reference)
import jax
import jax.numpy as jnp
from jax.experimental import pallas as pl


def kernel(x, y, target_lengths):
    raise NotImplementedError("write your pallas kernel here")


# speedup vs baseline: 11.6129x; 11.6129x over previous
import functools

import jax
import jax.numpy as jnp
from jax import lax
from jax.experimental import pallas as pl
from jax.experimental.pallas import tpu as pltpu
from jax.experimental.pallas import tpu_sc as plsc

_C = 64
_BLANK = 63
_B = 16
_T = 1024
_HALF = _T // 2
_NV = _HALF // 16
_YLEN = 1600
_YV = _YLEN // 16


def _sc_body(x_hbm, y_hbm, tl_hbm, nk_out, yk_out,
             xv, yv, tlv, sv, hist, yhist, nks, yks):
    c = lax.axis_index("c")
    s = lax.axis_index("s")
    b = s
    h = c

    lane = lax.iota(jnp.int32, 16)
    ones = jnp.ones((16,), jnp.float32)
    zeros = jnp.zeros((16,), jnp.float32)

    def zero_body(i, _):
        hist[i, :] = zeros
        yhist[i, :] = zeros
        return 0
    lax.fori_loop(0, _C, zero_body, 0)

    pltpu.sync_copy(x_hbm.at[b, :, pl.ds(h * _HALF, _HALF)], xv)

    def t_body(j, _):
        base = j * 16
        best = xv[0, pl.ds(base, 16)]
        bidx = jnp.zeros((16,), jnp.int32)
        for cc in range(1, _C):
            v = xv[cc, pl.ds(base, 16)]
            m = v > best
            best = jnp.where(m, v, best)
            bidx = jnp.where(m, jnp.full((16,), cc, jnp.int32), bidx)
        plsc.addupdate_scatter(hist, [bidx, lane], ones)
        return 0
    lax.fori_loop(0, _NV, t_body, 0)

    def _lane_reduce(src, dst):
        for g in range(_C // 16):
            cls = g * 16 + lane
            acc = zeros
            for k in range(16):
                col = jnp.full((16,), k, jnp.int32)
                acc = acc + plsc.load_gather(src, [cls, col])
            dst[pl.ds(g * 16, 16)] = acc

    _lane_reduce(hist, nks)
    pltpu.sync_copy(nks, nk_out.at[h, b])

    @pl.when(h == 0)
    def _y_hist():
        pltpu.sync_copy(y_hbm, yv)
        pltpu.sync_copy(tl_hbm, tlv)
        tl = tlv[...]
        incl = plsc.cumsum(tl)
        sv[...] = incl - tl
        bsplat = jnp.full((16,), b, jnp.int32)
        start = plsc.load_gather(sv, [bsplat])
        end = start + plsc.load_gather(tlv, [bsplat])

        def y_body(k, _):
            t0 = k * 16
            yy = yv[pl.ds(t0, 16)]
            t = t0 + lane
            m = (t >= start) & (t < end)
            plsc.addupdate_scatter(yhist, [yy, lane], ones, mask=m)
            return 0
        lax.fori_loop(0, _YV, y_body, 0)

        _lane_reduce(yhist, yks)
        pltpu.sync_copy(yks, yk_out.at[b])


_sc_counts = functools.partial(
    pl.kernel,
    mesh=plsc.VectorSubcoreMesh(core_axis_name="c", subcore_axis_name="s"),
    compiler_params=pltpu.CompilerParams(needs_layout_passes=False),
    out_type=[
        jax.ShapeDtypeStruct((2, _B, _C), jnp.float32),
        jax.ShapeDtypeStruct((_B, _C), jnp.float32),
    ],
    scratch_types=[
        pltpu.VMEM((_C, _HALF), jnp.float32),
        pltpu.VMEM((_YLEN,), jnp.int32),
        pltpu.VMEM((16,), jnp.int32),
        pltpu.VMEM((16,), jnp.int32),
        pltpu.VMEM((_C, 16), jnp.float32),
        pltpu.VMEM((_C, 16), jnp.float32),
        pltpu.VMEM((_C,), jnp.float32),
        pltpu.VMEM((_C,), jnp.float32),
    ],
)(_sc_body)


def _tc_loss_body(nk_ref, yk_ref, out_ref):
    nk = nk_ref[0] + nk_ref[1]
    yk = yk_ref[...]
    mask = yk != 0.0
    denom_n = jnp.sum(jnp.where(mask, nk, 0.0), axis=1) - nk[:, _BLANK]
    denom_y = jnp.sum(yk, axis=1) - yk[:, _BLANK]
    n_p = jnp.clip(nk / denom_n[:, None], 1e-5)
    y_p = yk / denom_y[:, None]
    m = (n_p + y_p) / 2.0
    kl1 = jnp.sum(jnp.where(mask, n_p * jnp.log(n_p / m), 0.0), axis=1)
    kl2 = jnp.sum(jnp.where(mask, y_p * jnp.log(y_p / m), 0.0), axis=1)
    out_ref[...] = jnp.full((1, 1), 1.0, jnp.float32) * jnp.mean(kl1 + kl2)


def kernel(x, y, target_lengths):
    nk, yk = _sc_counts(x, y, target_lengths.astype(jnp.int32))
    loss = pl.pallas_call(
        _tc_loss_body,
        out_shape=jax.ShapeDtypeStruct((1, 1), jnp.float32),
    )(nk, yk)
    return loss[0, 0]


# speedup vs baseline: 12.4572x; 1.0727x over previous
import functools

import jax
import jax.numpy as jnp
from jax import lax
from jax.experimental import pallas as pl
from jax.experimental.pallas import tpu as pltpu
from jax.experimental.pallas import tpu_sc as plsc

_C = 64
_BLANK = 63
_B = 16
_T = 1024
_HALF = _T // 2
_NV = _HALF // 16
_YLEN = 1600
_YV = _YLEN // 16


def _sc_body(x_hbm, y_hbm, tl_hbm, nk_out, yk_out,
             xv, yv, tlv, sv, hist, yhist, nks, yks):
    c = lax.axis_index("c")
    s = lax.axis_index("s")
    b = s
    h = c

    lane = lax.iota(jnp.int32, 16)
    ones = jnp.ones((16,), jnp.float32)
    zeros = jnp.zeros((16,), jnp.float32)

    def zero_body(i, _):
        hist[i, :] = zeros
        yhist[i, :] = zeros
        return 0
    lax.fori_loop(0, _C, zero_body, 0)

    pltpu.sync_copy(x_hbm.at[b, :, pl.ds(h * _HALF, _HALF)], xv)

    def t_body(j, _):
        base = j * 16
        bests, bidxs = [], []
        for k in range(8):
            c0 = k * 8
            best = xv[c0, pl.ds(base, 16)]
            bidx = jnp.full((16,), c0, jnp.int32)
            for cc in range(c0 + 1, c0 + 8):
                v = xv[cc, pl.ds(base, 16)]
                m = v > best
                best = jnp.where(m, v, best)
                bidx = jnp.where(m, jnp.full((16,), cc, jnp.int32), bidx)
            bests.append(best)
            bidxs.append(bidx)
        best, bidx = bests[0], bidxs[0]
        for k in range(1, 8):
            m = bests[k] > best
            best = jnp.where(m, bests[k], best)
            bidx = jnp.where(m, bidxs[k], bidx)
        plsc.addupdate_scatter(hist, [bidx, lane], ones)
        return 0
    lax.fori_loop(0, _NV, t_body, 0)

    def _lane_reduce(src, dst):
        for g in range(_C // 16):
            cls = g * 16 + lane
            acc = zeros
            for k in range(16):
                col = jnp.full((16,), k, jnp.int32)
                acc = acc + plsc.load_gather(src, [cls, col])
            dst[pl.ds(g * 16, 16)] = acc

    _lane_reduce(hist, nks)
    pltpu.sync_copy(nks, nk_out.at[h, b])

    @pl.when(h == (b % 2))
    def _y_hist():
        pltpu.sync_copy(y_hbm, yv)
        pltpu.sync_copy(tl_hbm, tlv)
        tl = tlv[...]
        incl = plsc.cumsum(tl)
        sv[...] = incl - tl
        bsplat = jnp.full((16,), b, jnp.int32)
        start = plsc.load_gather(sv, [bsplat])
        end = start + plsc.load_gather(tlv, [bsplat])

        def y_body(k, _):
            t0 = k * 16
            yy = yv[pl.ds(t0, 16)]
            t = t0 + lane
            m = (t >= start) & (t < end)
            plsc.addupdate_scatter(yhist, [yy, lane], ones, mask=m)
            return 0
        lax.fori_loop(0, _YV, y_body, 0)

        _lane_reduce(yhist, yks)
        pltpu.sync_copy(yks, yk_out.at[b])


_sc_counts = functools.partial(
    pl.kernel,
    mesh=plsc.VectorSubcoreMesh(core_axis_name="c", subcore_axis_name="s"),
    compiler_params=pltpu.CompilerParams(needs_layout_passes=False),
    out_type=[
        jax.ShapeDtypeStruct((2, _B, _C), jnp.float32),
        jax.ShapeDtypeStruct((_B, _C), jnp.float32),
    ],
    scratch_types=[
        pltpu.VMEM((_C, _HALF), jnp.float32),
        pltpu.VMEM((_YLEN,), jnp.int32),
        pltpu.VMEM((16,), jnp.int32),
        pltpu.VMEM((16,), jnp.int32),
        pltpu.VMEM((_C, 16), jnp.float32),
        pltpu.VMEM((_C, 16), jnp.float32),
        pltpu.VMEM((_C,), jnp.float32),
        pltpu.VMEM((_C,), jnp.float32),
    ],
)(_sc_body)


def _tc_loss_body(nk_ref, yk_ref, out_ref):
    nk = nk_ref[0] + nk_ref[1]
    yk = yk_ref[...]
    mask = yk != 0.0
    denom_n = jnp.sum(jnp.where(mask, nk, 0.0), axis=1) - nk[:, _BLANK]
    denom_y = jnp.sum(yk, axis=1) - yk[:, _BLANK]
    n_p = jnp.clip(nk / denom_n[:, None], 1e-5)
    y_p = yk / denom_y[:, None]
    m = (n_p + y_p) / 2.0
    kl1 = jnp.sum(jnp.where(mask, n_p * jnp.log(n_p / m), 0.0), axis=1)
    kl2 = jnp.sum(jnp.where(mask, y_p * jnp.log(y_p / m), 0.0), axis=1)
    out_ref[...] = jnp.full((1, 1), 1.0, jnp.float32) * jnp.mean(kl1 + kl2)


def kernel(x, y, target_lengths):
    nk, yk = _sc_counts(x, y, target_lengths.astype(jnp.int32))
    loss = pl.pallas_call(
        _tc_loss_body,
        out_shape=jax.ShapeDtypeStruct((1, 1), jnp.float32),
    )(nk, yk)
    return loss[0, 0]
